# baseline (speedup 1.0000x reference)
import functools
import math

import jax
import jax.numpy as jnp
from jax import lax
from jax.experimental import pallas as pl
from jax.experimental.pallas import tpu as pltpu
from jax.experimental.pallas import tpu_sc as plsc

D = 512
M = 8192
NB = 1024
TOPK = 8
LR_FAST = 1.0
LR_DEEP = 0.1
DECAY = 0.9995
MT = 512
CW = 512

NW = 32
BPW = NB // NW
SLAB = M // NW

_HIGH = jax.lax.Precision.HIGHEST
NEG = -3.0e38


def _layernorm(x, g, b):
    mu = jnp.mean(x, axis=-1, keepdims=True)
    var = jnp.mean((x - mu) ** 2, axis=-1, keepdims=True)
    return (x - mu) * jax.lax.rsqrt(var + 1e-5) * g + b


def _mamba_body(x_ref, w10, b10, w20, b20, g0, be0, w11, b11, w21, b21, g1,
                be1, o_ref):
    h = x_ref[...]
    for (w1, b1, w2, b2, g, be) in ((w10, b10, w20, b20, g0, be0),
                                    (w11, b11, w21, b21, g1, be1)):
        a = jnp.dot(h, w1[...],
                    preferred_element_type=jnp.float32) + b1[...]
        a = jax.nn.gelu(a)
        hh = jnp.dot(a, w2[...],
                     preferred_element_type=jnp.float32) + b2[...]
        hh = _layernorm(hh, g[...], be[...])
        h = h + hh
    o_ref[...] = h


def _mamba(xf, W1_0, b1_0, W2_0, b2_0, g_0, be_0, W1_1, b1_1, W2_1, b2_1,
           g_1, be_1):
    n = xf.shape[0]
    blk = 512 if n % 512 == 0 else n
    grid = (n // blk,)
    full = lambda shp: pl.BlockSpec(shp, lambda i: tuple(0 for _ in shp))
    return pl.pallas_call(
        _mamba_body,
        grid=grid,
        in_specs=[pl.BlockSpec((blk, D), lambda i: (i, 0))] + [
            full(w.shape) for w in (W1_0, b1_0, W2_0, b2_0, g_0, be_0,
                                    W1_1, b1_1, W2_1, b2_1, g_1, be_1)],
        out_specs=pl.BlockSpec((blk, D), lambda i: (i, 0)),
        out_shape=jax.ShapeDtypeStruct((n, D), jnp.float32),
        compiler_params=pltpu.CompilerParams(
            dimension_semantics=("arbitrary",)),
    )(xf, W1_0, b1_0, W2_0, b2_0, g_0, be_0, W1_1, b1_1, W2_1, b2_1, g_1,
      be_1)


def _select_body(h_ref, ak_ref, av_ref, s_ref, v_ref, idx_ref,
                 sim_ref, m1_ref, v8_ref, den_ref, *, nb, nmt):
    p = pl.program_id(0)
    j = pl.program_id(1)
    decp = s_ref[0]
    vscale = s_ref[1]
    invsq = s_ref[2]

    @pl.when(p == 0)
    def _phase_sim():
        sim_ref[:, pl.ds(j * MT, MT)] = invsq * jax.lax.dot_general(
            h_ref[...], decp * ak_ref[...], (((1,), (1,)), ((), ())),
            preferred_element_type=jnp.float32)

    @pl.when((p == 1) & (j == 0))
    def _phase_topk():
        nch = (nmt * MT) // CW

        def masked_max(thr):
            def body(c, cur):
                ch = sim_ref[:, pl.ds(c * CW, CW)]
                ch = jnp.where(ch < thr, ch, NEG)
                return jnp.maximum(cur, jnp.max(ch, axis=1, keepdims=True))
            return jax.lax.fori_loop(0, nch, body,
                                     jnp.full((nb, 1), NEG, jnp.float32))

        m1 = masked_max(jnp.full((nb, 1), 3.0e38, jnp.float32))
        thr = jax.lax.fori_loop(0, TOPK - 1, lambda k, t: masked_max(t), m1)
        m1_ref[...] = m1
        v8_ref[...] = thr

        def argbody(c, cur):
            ch = sim_ref[:, pl.ds(c * CW, CW)]
            io = jax.lax.broadcasted_iota(jnp.int32, (nb, CW), 1) + c * CW
            cand = jnp.min(jnp.where(ch == m1, io, jnp.int32(2 ** 30)),
                           axis=1, keepdims=True)
            return jnp.minimum(cur, cand)
        idx_ref[...] = jax.lax.fori_loop(
            0, nch, argbody, jnp.full((nb, 1), 2 ** 30, jnp.int32))

    @pl.when(p == 1)
    def _phase_read():
        simt = sim_ref[:, pl.ds(j * MT, MT)]
        pmat = jnp.where(simt >= v8_ref[...],
                         jnp.exp(simt - m1_ref[...]), 0.0)
        pv = jax.lax.dot_general(pmat, av_ref[...], (((1,), (0,)), ((), ())),
                                 precision=_HIGH,
                                 preferred_element_type=jnp.float32)
        dloc = jnp.sum(pmat, axis=1, keepdims=True)

        @pl.when(j == 0)
        def _init():
            v_ref[...] = pv
            den_ref[...] = dloc

        @pl.when(j > 0)
        def _acc():
            v_ref[...] += pv
            den_ref[...] += dloc

        @pl.when(j == nmt - 1)
        def _fin():
            v_ref[...] = v_ref[...] * (vscale / den_ref[...])


def _select(h_t, a_k, a_v, scal):
    nb = h_t.shape[0]
    m = a_k.shape[0]
    nmt = m // MT
    body = functools.partial(_select_body, nb=nb, nmt=nmt)
    return pl.pallas_call(
        body,
        grid=(2, nmt),
        in_specs=[
            pl.BlockSpec((nb, D), lambda p, j: (0, 0)),
            pl.BlockSpec((MT, D),
                         lambda p, j: (jnp.where(p == 0, j, nmt - 1), 0)),
            pl.BlockSpec((MT, D), lambda p, j: (jnp.where(p == 1, j, 0), 0)),
            pl.BlockSpec(memory_space=pltpu.SMEM),
        ],
        out_specs=[
            pl.BlockSpec((nb, D), lambda p, j: (0, 0)),
            pl.BlockSpec((nb, 1), lambda p, j: (0, 0)),
        ],
        out_shape=[
            jax.ShapeDtypeStruct((nb, D), jnp.float32),
            jax.ShapeDtypeStruct((nb, 1), jnp.int32),
        ],
        scratch_shapes=[
            pltpu.VMEM((nb, m), jnp.float32),
            pltpu.VMEM((nb, 1), jnp.float32),
            pltpu.VMEM((nb, 1), jnp.float32),
            pltpu.VMEM((nb, 1), jnp.float32),
        ],
        compiler_params=pltpu.CompilerParams(
            dimension_semantics=("arbitrary", "arbitrary"),
            vmem_limit_bytes=120 * 1024 * 1024),
    )(h_t, a_k, a_v, scal)


def _fused_upd_body(h_ref, vf_ref, vd_ref, wf_ref, bf_ref, g_ref, b_ref,
                    gkf_ref, gvf_ref, gkd_ref, gvd_ref,
                    icf_ref, irf_ref, icd_ref, ird_ref, s_ref,
                    out_ref, rkf_ref, rvf_ref, rkd_ref, rvd_ref):
    h = h_ref[...]
    v = 0.5 * (vf_ref[...] + vd_ref[...])
    fused = (jnp.dot(h, wf_ref[:D, :],
                     preferred_element_type=jnp.float32)
             + jnp.dot(v, wf_ref[D:, :],
                       preferred_element_type=jnp.float32) + bf_ref[...])
    fused = _layernorm(fused + h, g_ref[...], b_ref[...])
    out_ref[...] = fused

    for (ic, ir, gk, gv, rk, rv, lr_eff, c_eff) in (
            (icf_ref, irf_ref, gkf_ref, gvf_ref, rkf_ref, rvf_ref,
             s_ref[0], s_ref[1]),
            (icd_ref, ird_ref, gkd_ref, gvd_ref, rkd_ref, rvd_ref,
             s_ref[2], s_ref[3])):
        z = (ic[...] == ir[...]).astype(jnp.float32)
        cnt = jnp.sum(z, axis=1, keepdims=True)
        aggk = jnp.dot(z, h, precision=_HIGH,
                       preferred_element_type=jnp.float32)
        aggv = jnp.dot(z, fused, precision=_HIGH,
                       preferred_element_type=jnp.float32)
        fac = 1.0 - lr_eff * cnt
        rk[...] = gk[...] * fac + c_eff * aggk
        rv[...] = gv[...] * fac + c_eff * aggv


def _fused_upd(h_t, v_f, v_d, Wf, bf, g_ln, b_ln, gkf, gvf, gkd, gvd,
               icf, irf, icd, ird, scal):
    nb = h_t.shape[0]
    full = lambda shp: pl.BlockSpec(shp, lambda: tuple(0 for _ in shp))
    return pl.pallas_call(
        _fused_upd_body,
        in_specs=[full((nb, D)), full((nb, D)), full((nb, D)),
                  full((2 * D, D)), full((D,)), full((D,)), full((D,)),
                  full((nb, D)), full((nb, D)), full((nb, D)), full((nb, D)),
                  full((nb, 1)), full((1, nb)), full((nb, 1)), full((1, nb)),
                  pl.BlockSpec(memory_space=pltpu.SMEM)],
        out_specs=[full((nb, D))] * 5,
        out_shape=[jax.ShapeDtypeStruct((nb, D), jnp.float32)] * 5,
        compiler_params=pltpu.CompilerParams(
            vmem_limit_bytes=100 * 1024 * 1024),
    )(h_t, v_f, v_d, Wf, bf, g_ln, b_ln, gkf, gvf, gkd, gvd,
      icf, irf, icd, ird, scal)


def _fused_only_body(h_ref, vf_ref, vd_ref, wf_ref, bf_ref, g_ref, b_ref,
                     out_ref):
    h = h_ref[...]
    v = 0.5 * (vf_ref[...] + vd_ref[...])
    fused = (jnp.dot(h, wf_ref[:D, :],
                     preferred_element_type=jnp.float32)
             + jnp.dot(v, wf_ref[D:, :],
                       preferred_element_type=jnp.float32) + bf_ref[...])
    out_ref[...] = _layernorm(fused + h, g_ref[...], b_ref[...])


def _fused_only(h_t, v_f, v_d, Wf, bf, g_ln, b_ln):
    nb = h_t.shape[0]
    full = lambda shp: pl.BlockSpec(shp, lambda: tuple(0 for _ in shp))
    return pl.pallas_call(
        _fused_only_body,
        in_specs=[full((nb, D)), full((nb, D)), full((nb, D)),
                  full((2 * D, D)), full((D,)), full((D,)), full((D,))],
        out_specs=full((nb, D)),
        out_shape=jax.ShapeDtypeStruct((nb, D), jnp.float32),
    )(h_t, v_f, v_d, Wf, bf, g_ln, b_ln)


def _sc_gather_body(akf, avf, akd, avd, idxf, idxd,
                    gkf, gvf, gkd, gvd, idxv, rows, sem):
    wid = lax.axis_index("s") * 2 + lax.axis_index("c")
    base = wid * BPW
    for idx_hbm, pairs in ((idxf, ((akf, gkf), (avf, gvf))),
                           (idxd, ((akd, gkd), (avd, gvd)))):
        pltpu.sync_copy(idx_hbm.at[pl.ds(base, BPW)], idxv)
        for tab, out in pairs:
            pltpu.async_copy(tab.at[idxv], rows, sem).wait()
            pltpu.sync_copy(rows, out.at[pl.ds(base, BPW)])


@functools.lru_cache(maxsize=None)
def _sc_gather_kernel():
    return pl.kernel(
        _sc_gather_body,
        out_type=[jax.ShapeDtypeStruct((NB, D), jnp.float32)] * 4,
        mesh=plsc.VectorSubcoreMesh(core_axis_name="c", subcore_axis_name="s",
                                    num_cores=2, num_subcores=16),
        scratch_types=[
            pltpu.VMEM((BPW,), jnp.int32),
            pltpu.VMEM((BPW, D), jnp.float32),
            pltpu.SemaphoreType.DMA,
        ],
    )


def _sc_gather(*args):
    return _sc_gather_kernel()(*args)


def _sc_scatter_body(akf, avf, akd, avd, idxf, idxd, rkf, rvf, rkd, rvd,
                     okf, ovf, okd, ovd, idq, sem, csem):
    wid = lax.axis_index("s") * 2 + lax.axis_index("c")
    lo = wid * SLAB
    slabs = ((akf, okf), (avf, ovf), (akd, okd), (avd, ovd))
    cps = [pltpu.async_copy(src.at[pl.ds(lo, SLAB)],
                            dst.at[pl.ds(lo, SLAB)], csem)
           for src, dst in slabs]
    for cp in cps:
        cp.wait()

    def scan(idx_hbm, rk, rv, ok, ov):
        pltpu.sync_copy(idx_hbm, idq)

        def group(gi, tot):
            vec = idq[pl.ds(gi * 16, 16)]
            t = tot
            for k in range(16):
                r = vec[k]
                hit = (r >= lo) & (r < lo + SLAB)

                @pl.when(hit)
                def _():
                    pltpu.async_copy(rk.at[pl.ds(gi * 16 + k, 1)],
                                     ok.at[pl.ds(r, 1)], sem)
                    pltpu.async_copy(rv.at[pl.ds(gi * 16 + k, 1)],
                                     ov.at[pl.ds(r, 1)], sem)
                t = t + jnp.where(hit, 1, 0)
            return t
        n = lax.fori_loop(0, NB // 16, group, jnp.int32(0))

        def drain(i, carry):
            pltpu.make_async_copy(rk.at[pl.ds(0, 1)],
                                  ok.at[pl.ds(lo, 1)], sem).wait()
            return carry
        lax.fori_loop(0, 2 * n, drain, jnp.int32(0))

    scan(idxf, rkf, rvf, okf, ovf)
    scan(idxd, rkd, rvd, okd, ovd)


@functools.lru_cache(maxsize=None)
def _sc_scatter_kernel():
    return pl.kernel(
        _sc_scatter_body,
        out_type=[jax.ShapeDtypeStruct((M, D), jnp.float32)] * 4,
        mesh=plsc.VectorSubcoreMesh(core_axis_name="c", subcore_axis_name="s",
                                    num_cores=2, num_subcores=16),
        scratch_types=[
            pltpu.VMEM((NB,), jnp.int32),
            pltpu.SemaphoreType.DMA,
            pltpu.SemaphoreType.DMA,
        ],
    )


def _sc_scatter(*args):
    return _sc_scatter_kernel()(*args)


def kernel(x, write_mask, W1_0, b1_0, W2_0, b2_0, g_0, be_0, W1_1, b1_1,
           W2_1, b2_1, g_1, be_1, Wf, bf, g_ln, b_ln, K_fast, V_fast,
           K_deep, V_deep):
    B, T, d = x.shape
    xf = x.reshape(B * T, d)
    h = _mamba(xf, W1_0, b1_0, W2_0, b2_0, g_0, be_0, W1_1, b1_1, W2_1,
               b2_1, g_1, be_1).reshape(B, T, d)

    anyb = jnp.any(write_mask, axis=0).astype(jnp.float32)
    invsq = jnp.float32(1.0 / math.sqrt(d))

    akf, avf, akd, avd = K_fast, V_fast, K_deep, V_deep
    p_t = jnp.float32(0.0)
    outs = []
    for t in range(T):
        h_t = h[:, t, :]
        decp = DECAY ** p_t
        scal = jnp.stack([decp, decp, invsq])
        vf, idxf = _select(h_t, akf, avf, scal)
        vd, idxd = _select(h_t, akd, avd, scal)
        if t + 1 < T:
            a_t = anyb[t]
            cscale = a_t / (decp * DECAY)
            uscal = jnp.stack([LR_FAST * a_t, LR_FAST * cscale,
                               LR_DEEP * a_t, LR_DEEP * cscale])
            gkf, gvf, gkd, gvd = _sc_gather(
                akf, avf, akd, avd, idxf.reshape(B), idxd.reshape(B))
            out_t, rkf, rvf, rkd, rvd = _fused_upd(
                h_t, vf, vd, Wf, bf, g_ln, b_ln, gkf, gvf, gkd, gvd,
                idxf, idxf.reshape(1, B), idxd, idxd.reshape(1, B), uscal)
            akf, avf, akd, avd = _sc_scatter(
                akf, avf, akd, avd, idxf.reshape(B), idxd.reshape(B),
                rkf, rvf, rkd, rvd)
            p_t = p_t + a_t
        else:
            out_t = _fused_only(h_t, vf, vd, Wf, bf, g_ln, b_ln)
        outs.append(out_t)
    return jnp.stack(outs, axis=1)


# speedup vs baseline: 6.1681x; 6.1681x over previous
import functools
import math

import jax
import jax.numpy as jnp
from jax import lax
from jax.experimental import pallas as pl
from jax.experimental.pallas import tpu as pltpu
from jax.experimental.pallas import tpu_sc as plsc

D = 512
M = 8192
NB = 1024
TOPK = 8
LR_FAST = 1.0
LR_DEEP = 0.1
DECAY = 0.9995
MT = 512
CW = 512

NW = 32
BPW = NB // NW
SLAB = M // NW

_HIGH = jax.lax.Precision.HIGHEST
NEG = -3.0e38


def _layernorm(x, g, b):
    mu = jnp.mean(x, axis=-1, keepdims=True)
    var = jnp.mean((x - mu) ** 2, axis=-1, keepdims=True)
    return (x - mu) * jax.lax.rsqrt(var + 1e-5) * g + b


def _mamba_body(x_ref, w10, b10, w20, b20, g0, be0, w11, b11, w21, b21, g1,
                be1, o_ref):
    h = x_ref[...]
    for (w1, b1, w2, b2, g, be) in ((w10, b10, w20, b20, g0, be0),
                                    (w11, b11, w21, b21, g1, be1)):
        a = jnp.dot(h, w1[...],
                    preferred_element_type=jnp.float32) + b1[...]
        a = jax.nn.gelu(a)
        hh = jnp.dot(a, w2[...],
                     preferred_element_type=jnp.float32) + b2[...]
        hh = _layernorm(hh, g[...], be[...])
        h = h + hh
    o_ref[...] = h


def _mamba(xf, W1_0, b1_0, W2_0, b2_0, g_0, be_0, W1_1, b1_1, W2_1, b2_1,
           g_1, be_1):
    n = xf.shape[0]
    blk = 512 if n % 512 == 0 else n
    grid = (n // blk,)
    full = lambda shp: pl.BlockSpec(shp, lambda i: tuple(0 for _ in shp))
    return pl.pallas_call(
        _mamba_body,
        grid=grid,
        in_specs=[pl.BlockSpec((blk, D), lambda i: (i, 0))] + [
            full(w.shape) for w in (W1_0, b1_0, W2_0, b2_0, g_0, be_0,
                                    W1_1, b1_1, W2_1, b2_1, g_1, be_1)],
        out_specs=pl.BlockSpec((blk, D), lambda i: (i, 0)),
        out_shape=jax.ShapeDtypeStruct((n, D), jnp.float32),
        compiler_params=pltpu.CompilerParams(
            dimension_semantics=("arbitrary",)),
    )(xf, W1_0, b1_0, W2_0, b2_0, g_0, be_0, W1_1, b1_1, W2_1, b2_1, g_1,
      be_1)


def _select_body(h_ref, ak_ref, av_ref, s_ref, v_ref, idx_ref,
                 sim_ref, m1_ref, v8_ref, den_ref, *, nb, nmt):
    p = pl.program_id(0)
    j = pl.program_id(1)
    decp = s_ref[0]
    vscale = s_ref[1]
    invsq = s_ref[2]

    @pl.when(p == 0)
    def _phase_sim():
        sim_ref[:, pl.ds(j * MT, MT)] = invsq * jax.lax.dot_general(
            h_ref[...], decp * ak_ref[...], (((1,), (1,)), ((), ())),
            preferred_element_type=jnp.float32)

    @pl.when((p == 1) & (j == 0))
    def _phase_topk():
        nch = (nmt * MT) // CW

        def masked_max(thr):
            def body(c, cur):
                ch = sim_ref[:, pl.ds(c * CW, CW)]
                ch = jnp.where(ch < thr, ch, NEG)
                return jnp.maximum(cur, jnp.max(ch, axis=1, keepdims=True))
            return jax.lax.fori_loop(0, nch, body,
                                     jnp.full((nb, 1), NEG, jnp.float32))

        m1 = masked_max(jnp.full((nb, 1), 3.0e38, jnp.float32))
        thr = jax.lax.fori_loop(0, TOPK - 1, lambda k, t: masked_max(t), m1)
        m1_ref[...] = m1
        v8_ref[...] = thr

        def argbody(c, cur):
            ch = sim_ref[:, pl.ds(c * CW, CW)]
            io = jax.lax.broadcasted_iota(jnp.int32, (nb, CW), 1) + c * CW
            cand = jnp.min(jnp.where(ch == m1, io, jnp.int32(2 ** 30)),
                           axis=1, keepdims=True)
            return jnp.minimum(cur, cand)
        idx_ref[...] = jax.lax.fori_loop(
            0, nch, argbody, jnp.full((nb, 1), 2 ** 30, jnp.int32))

    @pl.when(p == 1)
    def _phase_read():
        simt = sim_ref[:, pl.ds(j * MT, MT)]
        pmat = jnp.where(simt >= v8_ref[...],
                         jnp.exp(simt - m1_ref[...]), 0.0)
        pv = jax.lax.dot_general(pmat, av_ref[...], (((1,), (0,)), ((), ())),
                                 preferred_element_type=jnp.float32)
        dloc = jnp.sum(pmat, axis=1, keepdims=True)

        @pl.when(j == 0)
        def _init():
            v_ref[...] = pv
            den_ref[...] = dloc

        @pl.when(j > 0)
        def _acc():
            v_ref[...] += pv
            den_ref[...] += dloc

        @pl.when(j == nmt - 1)
        def _fin():
            v_ref[...] = v_ref[...] * (vscale / den_ref[...])


def _select(h_t, a_k, a_v, scal):
    nb = h_t.shape[0]
    m = a_k.shape[0]
    nmt = m // MT
    body = functools.partial(_select_body, nb=nb, nmt=nmt)
    return pl.pallas_call(
        body,
        grid=(2, nmt),
        in_specs=[
            pl.BlockSpec((nb, D), lambda p, j: (0, 0)),
            pl.BlockSpec((MT, D),
                         lambda p, j: (jnp.where(p == 0, j, nmt - 1), 0)),
            pl.BlockSpec((MT, D), lambda p, j: (jnp.where(p == 1, j, 0), 0)),
            pl.BlockSpec(memory_space=pltpu.SMEM),
        ],
        out_specs=[
            pl.BlockSpec((nb, D), lambda p, j: (0, 0)),
            pl.BlockSpec((nb, 1), lambda p, j: (0, 0)),
        ],
        out_shape=[
            jax.ShapeDtypeStruct((nb, D), jnp.float32),
            jax.ShapeDtypeStruct((nb, 1), jnp.int32),
        ],
        scratch_shapes=[
            pltpu.VMEM((nb, m), jnp.float32),
            pltpu.VMEM((nb, 1), jnp.float32),
            pltpu.VMEM((nb, 1), jnp.float32),
            pltpu.VMEM((nb, 1), jnp.float32),
        ],
        compiler_params=pltpu.CompilerParams(
            dimension_semantics=("arbitrary", "arbitrary"),
            vmem_limit_bytes=120 * 1024 * 1024),
    )(h_t, a_k, a_v, scal)


def _fused_upd_body(h_ref, vf_ref, vd_ref, wf_ref, bf_ref, g_ref, b_ref,
                    gkf_ref, gvf_ref, gkd_ref, gvd_ref,
                    icf_ref, irf_ref, icd_ref, ird_ref, s_ref,
                    out_ref, rkf_ref, rvf_ref, rkd_ref, rvd_ref):
    h = h_ref[...]
    v = 0.5 * (vf_ref[...] + vd_ref[...])
    fused = (jnp.dot(h, wf_ref[:D, :],
                     preferred_element_type=jnp.float32)
             + jnp.dot(v, wf_ref[D:, :],
                       preferred_element_type=jnp.float32) + bf_ref[...])
    fused = _layernorm(fused + h, g_ref[...], b_ref[...])
    out_ref[...] = fused

    for (ic, ir, gk, gv, rk, rv, lr_eff, c_eff) in (
            (icf_ref, irf_ref, gkf_ref, gvf_ref, rkf_ref, rvf_ref,
             s_ref[0], s_ref[1]),
            (icd_ref, ird_ref, gkd_ref, gvd_ref, rkd_ref, rvd_ref,
             s_ref[2], s_ref[3])):
        z = (ic[...] == ir[...]).astype(jnp.float32)
        cnt = jnp.sum(z, axis=1, keepdims=True)
        aggk = jnp.dot(z, h, precision=_HIGH,
                       preferred_element_type=jnp.float32)
        aggv = jnp.dot(z, fused, precision=_HIGH,
                       preferred_element_type=jnp.float32)
        fac = 1.0 - lr_eff * cnt
        rk[...] = gk[...] * fac + c_eff * aggk
        rv[...] = gv[...] * fac + c_eff * aggv


def _fused_upd(h_t, v_f, v_d, Wf, bf, g_ln, b_ln, gkf, gvf, gkd, gvd,
               icf, irf, icd, ird, scal):
    nb = h_t.shape[0]
    full = lambda shp: pl.BlockSpec(shp, lambda: tuple(0 for _ in shp))
    return pl.pallas_call(
        _fused_upd_body,
        in_specs=[full((nb, D)), full((nb, D)), full((nb, D)),
                  full((2 * D, D)), full((D,)), full((D,)), full((D,)),
                  full((nb, D)), full((nb, D)), full((nb, D)), full((nb, D)),
                  full((nb, 1)), full((1, nb)), full((nb, 1)), full((1, nb)),
                  pl.BlockSpec(memory_space=pltpu.SMEM)],
        out_specs=[full((nb, D))] * 5,
        out_shape=[jax.ShapeDtypeStruct((nb, D), jnp.float32)] * 5,
        compiler_params=pltpu.CompilerParams(
            vmem_limit_bytes=100 * 1024 * 1024),
    )(h_t, v_f, v_d, Wf, bf, g_ln, b_ln, gkf, gvf, gkd, gvd,
      icf, irf, icd, ird, scal)


def _fused_only_body(h_ref, vf_ref, vd_ref, wf_ref, bf_ref, g_ref, b_ref,
                     out_ref):
    h = h_ref[...]
    v = 0.5 * (vf_ref[...] + vd_ref[...])
    fused = (jnp.dot(h, wf_ref[:D, :],
                     preferred_element_type=jnp.float32)
             + jnp.dot(v, wf_ref[D:, :],
                       preferred_element_type=jnp.float32) + bf_ref[...])
    out_ref[...] = _layernorm(fused + h, g_ref[...], b_ref[...])


def _fused_only(h_t, v_f, v_d, Wf, bf, g_ln, b_ln):
    nb = h_t.shape[0]
    full = lambda shp: pl.BlockSpec(shp, lambda: tuple(0 for _ in shp))
    return pl.pallas_call(
        _fused_only_body,
        in_specs=[full((nb, D)), full((nb, D)), full((nb, D)),
                  full((2 * D, D)), full((D,)), full((D,)), full((D,))],
        out_specs=full((nb, D)),
        out_shape=jax.ShapeDtypeStruct((nb, D), jnp.float32),
    )(h_t, v_f, v_d, Wf, bf, g_ln, b_ln)


def _sc_gather_body(akf, avf, akd, avd, idxf, idxd,
                    gkf, gvf, gkd, gvd, idxv, rows, sem):
    wid = lax.axis_index("s") * 2 + lax.axis_index("c")
    base = wid * BPW
    for idx_hbm, pairs in ((idxf, ((akf, gkf), (avf, gvf))),
                           (idxd, ((akd, gkd), (avd, gvd)))):
        pltpu.sync_copy(idx_hbm.at[pl.ds(base, BPW)], idxv)
        for tab, out in pairs:
            pltpu.async_copy(tab.at[idxv], rows, sem).wait()
            pltpu.sync_copy(rows, out.at[pl.ds(base, BPW)])


@functools.lru_cache(maxsize=None)
def _sc_gather_kernel():
    return pl.kernel(
        _sc_gather_body,
        out_type=[jax.ShapeDtypeStruct((NB, D), jnp.float32)] * 4,
        mesh=plsc.VectorSubcoreMesh(core_axis_name="c", subcore_axis_name="s",
                                    num_cores=2, num_subcores=16),
        scratch_types=[
            pltpu.VMEM((BPW,), jnp.int32),
            pltpu.VMEM((BPW, D), jnp.float32),
            pltpu.SemaphoreType.DMA,
        ],
    )


def _sc_gather(*args):
    return _sc_gather_kernel()(*args)


def _scatter_tc_body(idxf, idxd, rkf, rvf, rkd, rvd, akf, avf, akd, avd,
                     okf, ovf, okd, ovd, sem, *, nb):
    def scan(idxref, rk, rv, ok, ov):
        def it(i, c):
            r = idxref[i]
            pltpu.make_async_copy(rk.at[pl.ds(i, 1)],
                                  ok.at[pl.ds(r, 1)], sem).start()
            pltpu.make_async_copy(rv.at[pl.ds(i, 1)],
                                  ov.at[pl.ds(r, 1)], sem).start()
            return c
        lax.fori_loop(0, nb, it, jnp.int32(0))
    scan(idxf, rkf, rvf, okf, ovf)
    scan(idxd, rkd, rvd, okd, ovd)

    def drain(i, c):
        pltpu.make_async_copy(rkf.at[pl.ds(0, 1)],
                              okf.at[pl.ds(0, 1)], sem).wait()
        return c
    lax.fori_loop(0, 4 * nb, drain, jnp.int32(0))


def _sc_scatter(akf, avf, akd, avd, idxf, idxd, rkf, rvf, rkd, rvd):
    nb = rkf.shape[0]
    m = akf.shape[0]
    body = functools.partial(_scatter_tc_body, nb=nb)
    vspec = pl.BlockSpec((nb, D), lambda: (0, 0))
    ispec = pl.BlockSpec(memory_space=pltpu.SMEM)
    aspec = pl.BlockSpec(memory_space=pl.ANY)
    return pl.pallas_call(
        body,
        in_specs=[ispec, ispec, vspec, vspec, vspec, vspec,
                  aspec, aspec, aspec, aspec],
        out_specs=[aspec] * 4,
        out_shape=[jax.ShapeDtypeStruct((m, D), jnp.float32)] * 4,
        input_output_aliases={6: 0, 7: 1, 8: 2, 9: 3},
        scratch_shapes=[pltpu.SemaphoreType.DMA],
        compiler_params=pltpu.CompilerParams(
            vmem_limit_bytes=100 * 1024 * 1024),
    )(idxf, idxd, rkf, rvf, rkd, rvd, akf, avf, akd, avd)


def kernel(x, write_mask, W1_0, b1_0, W2_0, b2_0, g_0, be_0, W1_1, b1_1,
           W2_1, b2_1, g_1, be_1, Wf, bf, g_ln, b_ln, K_fast, V_fast,
           K_deep, V_deep):
    B, T, d = x.shape
    xf = x.reshape(B * T, d)
    h = _mamba(xf, W1_0, b1_0, W2_0, b2_0, g_0, be_0, W1_1, b1_1, W2_1,
               b2_1, g_1, be_1).reshape(B, T, d)

    anyb = jnp.any(write_mask, axis=0).astype(jnp.float32)
    invsq = jnp.float32(1.0 / math.sqrt(d))

    akf, avf, akd, avd = K_fast, V_fast, K_deep, V_deep
    p_t = jnp.float32(0.0)
    outs = []
    for t in range(T):
        h_t = h[:, t, :]
        decp = DECAY ** p_t
        scal = jnp.stack([decp, decp, invsq])
        vf, idxf = _select(h_t, akf, avf, scal)
        vd, idxd = _select(h_t, akd, avd, scal)
        if t + 1 < T:
            a_t = anyb[t]
            cscale = a_t / (decp * DECAY)
            uscal = jnp.stack([LR_FAST * a_t, LR_FAST * cscale,
                               LR_DEEP * a_t, LR_DEEP * cscale])
            gkf, gvf, gkd, gvd = _sc_gather(
                akf, avf, akd, avd, idxf.reshape(B), idxd.reshape(B))
            out_t, rkf, rvf, rkd, rvd = _fused_upd(
                h_t, vf, vd, Wf, bf, g_ln, b_ln, gkf, gvf, gkd, gvd,
                idxf, idxf.reshape(1, B), idxd, idxd.reshape(1, B), uscal)
            akf, avf, akd, avd = _sc_scatter(
                akf, avf, akd, avd, idxf.reshape(B), idxd.reshape(B),
                rkf, rvf, rkd, rvd)
            p_t = p_t + a_t
        else:
            out_t = _fused_only(h_t, vf, vd, Wf, bf, g_ln, b_ln)
        outs.append(out_t)
    return jnp.stack(outs, axis=1)


# speedup vs baseline: 6.2876x; 1.0194x over previous
import functools
import math

import jax
import jax.numpy as jnp
from jax import lax
from jax.experimental import pallas as pl
from jax.experimental.pallas import tpu as pltpu
from jax.experimental.pallas import tpu_sc as plsc

D = 512
M = 8192
NB = 1024
TOPK = 8
LR_FAST = 1.0
LR_DEEP = 0.1
DECAY = 0.9995
MT = 512
CW = 512

NW = 32
BPW = NB // NW
SLAB = M // NW

_HIGH = jax.lax.Precision.HIGHEST
NEG = -3.0e38


def _layernorm(x, g, b):
    mu = jnp.mean(x, axis=-1, keepdims=True)
    var = jnp.mean((x - mu) ** 2, axis=-1, keepdims=True)
    return (x - mu) * jax.lax.rsqrt(var + 1e-5) * g + b


def _mamba_body(x_ref, w10, b10, w20, b20, g0, be0, w11, b11, w21, b21, g1,
                be1, o_ref):
    h = x_ref[...]
    for (w1, b1, w2, b2, g, be) in ((w10, b10, w20, b20, g0, be0),
                                    (w11, b11, w21, b21, g1, be1)):
        a = jnp.dot(h, w1[...],
                    preferred_element_type=jnp.float32) + b1[...]
        a = jax.nn.gelu(a)
        hh = jnp.dot(a, w2[...],
                     preferred_element_type=jnp.float32) + b2[...]
        hh = _layernorm(hh, g[...], be[...])
        h = h + hh
    o_ref[...] = h


def _mamba(xf, W1_0, b1_0, W2_0, b2_0, g_0, be_0, W1_1, b1_1, W2_1, b2_1,
           g_1, be_1):
    n = xf.shape[0]
    blk = 512 if n % 512 == 0 else n
    grid = (n // blk,)
    full = lambda shp: pl.BlockSpec(shp, lambda i: tuple(0 for _ in shp))
    return pl.pallas_call(
        _mamba_body,
        grid=grid,
        in_specs=[pl.BlockSpec((blk, D), lambda i: (i, 0))] + [
            full(w.shape) for w in (W1_0, b1_0, W2_0, b2_0, g_0, be_0,
                                    W1_1, b1_1, W2_1, b2_1, g_1, be_1)],
        out_specs=pl.BlockSpec((blk, D), lambda i: (i, 0)),
        out_shape=jax.ShapeDtypeStruct((n, D), jnp.float32),
        compiler_params=pltpu.CompilerParams(
            dimension_semantics=("arbitrary",)),
    )(xf, W1_0, b1_0, W2_0, b2_0, g_0, be_0, W1_1, b1_1, W2_1, b2_1, g_1,
      be_1)


def _select_body(h_ref, ak_ref, av_ref, s_ref, v_ref, idx_ref,
                 sim_ref, m1_ref, v8_ref, den_ref, *, nb, nmt):
    p = pl.program_id(0)
    j = pl.program_id(1)
    decp = s_ref[0]
    vscale = s_ref[1]
    invsq = s_ref[2]

    @pl.when(p == 0)
    def _phase_sim():
        sim_ref[:, pl.ds(j * MT, MT)] = invsq * jax.lax.dot_general(
            h_ref[...], decp * ak_ref[...], (((1,), (1,)), ((), ())),
            preferred_element_type=jnp.float32)

    @pl.when((p == 1) & (j == 0))
    def _phase_topk():
        nch = (nmt * MT) // CW

        def masked_max(thr):
            def body(c, cur):
                ch = sim_ref[:, pl.ds(c * CW, CW)]
                ch = jnp.where(ch < thr, ch, NEG)
                return jnp.maximum(cur, jnp.max(ch, axis=1, keepdims=True))
            return jax.lax.fori_loop(0, nch, body,
                                     jnp.full((nb, 1), NEG, jnp.float32))

        m1 = masked_max(jnp.full((nb, 1), 3.0e38, jnp.float32))
        thr = jax.lax.fori_loop(0, TOPK - 1, lambda k, t: masked_max(t), m1)
        m1_ref[...] = m1
        v8_ref[...] = thr

        def argbody(c, cur):
            ch = sim_ref[:, pl.ds(c * CW, CW)]
            io = jax.lax.broadcasted_iota(jnp.int32, (nb, CW), 1) + c * CW
            cand = jnp.min(jnp.where(ch == m1, io, jnp.int32(2 ** 30)),
                           axis=1, keepdims=True)
            return jnp.minimum(cur, cand)
        idx_ref[...] = jax.lax.fori_loop(
            0, nch, argbody, jnp.full((nb, 1), 2 ** 30, jnp.int32))

    @pl.when(p == 1)
    def _phase_read():
        simt = sim_ref[:, pl.ds(j * MT, MT)]
        pmat = jnp.where(simt >= v8_ref[...],
                         jnp.exp(simt - m1_ref[...]), 0.0)
        pv = jax.lax.dot_general(pmat, av_ref[...], (((1,), (0,)), ((), ())),
                                 preferred_element_type=jnp.float32)
        dloc = jnp.sum(pmat, axis=1, keepdims=True)

        @pl.when(j == 0)
        def _init():
            v_ref[...] = pv
            den_ref[...] = dloc

        @pl.when(j > 0)
        def _acc():
            v_ref[...] += pv
            den_ref[...] += dloc

        @pl.when(j == nmt - 1)
        def _fin():
            v_ref[...] = v_ref[...] * (vscale / den_ref[...])


def _select(h_t, a_k, a_v, scal):
    nb = h_t.shape[0]
    m = a_k.shape[0]
    nmt = m // MT
    body = functools.partial(_select_body, nb=nb, nmt=nmt)
    return pl.pallas_call(
        body,
        grid=(2, nmt),
        in_specs=[
            pl.BlockSpec((nb, D), lambda p, j: (0, 0)),
            pl.BlockSpec((MT, D),
                         lambda p, j: (jnp.where(p == 0, j, nmt - 1), 0)),
            pl.BlockSpec((MT, D), lambda p, j: (jnp.where(p == 1, j, 0), 0)),
            pl.BlockSpec(memory_space=pltpu.SMEM),
        ],
        out_specs=[
            pl.BlockSpec((nb, D), lambda p, j: (0, 0)),
            pl.BlockSpec((nb, 1), lambda p, j: (0, 0)),
        ],
        out_shape=[
            jax.ShapeDtypeStruct((nb, D), jnp.float32),
            jax.ShapeDtypeStruct((nb, 1), jnp.int32),
        ],
        scratch_shapes=[
            pltpu.VMEM((nb, m), jnp.float32),
            pltpu.VMEM((nb, 1), jnp.float32),
            pltpu.VMEM((nb, 1), jnp.float32),
            pltpu.VMEM((nb, 1), jnp.float32),
        ],
        compiler_params=pltpu.CompilerParams(
            dimension_semantics=("arbitrary", "arbitrary"),
            vmem_limit_bytes=120 * 1024 * 1024),
    )(h_t, a_k, a_v, scal)


def _fused_upd_body(h_ref, vf_ref, vd_ref, wf_ref, bf_ref, g_ref, b_ref,
                    gkf_ref, gvf_ref, gkd_ref, gvd_ref,
                    icf_ref, irf_ref, icd_ref, ird_ref, s_ref,
                    out_ref, rkf_ref, rvf_ref, rkd_ref, rvd_ref):
    h = h_ref[...]
    v = 0.5 * (vf_ref[...] + vd_ref[...])
    fused = (jnp.dot(h, wf_ref[:D, :],
                     preferred_element_type=jnp.float32)
             + jnp.dot(v, wf_ref[D:, :],
                       preferred_element_type=jnp.float32) + bf_ref[...])
    fused = _layernorm(fused + h, g_ref[...], b_ref[...])
    out_ref[...] = fused

    for (ic, ir, gk, gv, rk, rv, lr_eff, c_eff) in (
            (icf_ref, irf_ref, gkf_ref, gvf_ref, rkf_ref, rvf_ref,
             s_ref[0], s_ref[1]),
            (icd_ref, ird_ref, gkd_ref, gvd_ref, rkd_ref, rvd_ref,
             s_ref[2], s_ref[3])):
        z = (ic[...] == ir[...]).astype(jnp.float32)
        cnt = jnp.sum(z, axis=1, keepdims=True)
        aggk = jnp.dot(z, h, precision=_HIGH,
                       preferred_element_type=jnp.float32)
        aggv = jnp.dot(z, fused, precision=_HIGH,
                       preferred_element_type=jnp.float32)
        fac = 1.0 - lr_eff * cnt
        rk[...] = gk[...] * fac + c_eff * aggk
        rv[...] = gv[...] * fac + c_eff * aggv


def _fused_upd(h_t, v_f, v_d, Wf, bf, g_ln, b_ln, gkf, gvf, gkd, gvd,
               icf, irf, icd, ird, scal):
    nb = h_t.shape[0]
    full = lambda shp: pl.BlockSpec(shp, lambda: tuple(0 for _ in shp))
    return pl.pallas_call(
        _fused_upd_body,
        in_specs=[full((nb, D)), full((nb, D)), full((nb, D)),
                  full((2 * D, D)), full((D,)), full((D,)), full((D,)),
                  full((nb, D)), full((nb, D)), full((nb, D)), full((nb, D)),
                  full((nb, 1)), full((1, nb)), full((nb, 1)), full((1, nb)),
                  pl.BlockSpec(memory_space=pltpu.SMEM)],
        out_specs=[full((nb, D))] * 5,
        out_shape=[jax.ShapeDtypeStruct((nb, D), jnp.float32)] * 5,
        compiler_params=pltpu.CompilerParams(
            vmem_limit_bytes=100 * 1024 * 1024),
    )(h_t, v_f, v_d, Wf, bf, g_ln, b_ln, gkf, gvf, gkd, gvd,
      icf, irf, icd, ird, scal)


def _fused_only_body(h_ref, vf_ref, vd_ref, wf_ref, bf_ref, g_ref, b_ref,
                     out_ref):
    h = h_ref[...]
    v = 0.5 * (vf_ref[...] + vd_ref[...])
    fused = (jnp.dot(h, wf_ref[:D, :],
                     preferred_element_type=jnp.float32)
             + jnp.dot(v, wf_ref[D:, :],
                       preferred_element_type=jnp.float32) + bf_ref[...])
    out_ref[...] = _layernorm(fused + h, g_ref[...], b_ref[...])


def _fused_only(h_t, v_f, v_d, Wf, bf, g_ln, b_ln):
    nb = h_t.shape[0]
    full = lambda shp: pl.BlockSpec(shp, lambda: tuple(0 for _ in shp))
    return pl.pallas_call(
        _fused_only_body,
        in_specs=[full((nb, D)), full((nb, D)), full((nb, D)),
                  full((2 * D, D)), full((D,)), full((D,)), full((D,))],
        out_specs=full((nb, D)),
        out_shape=jax.ShapeDtypeStruct((nb, D), jnp.float32),
    )(h_t, v_f, v_d, Wf, bf, g_ln, b_ln)


def _sc_gather_body(akf, avf, akd, avd, idxf, idxd,
                    gkf, gvf, gkd, gvd, idxvf, idxvd, rows, sem):
    wid = lax.axis_index("s") * 2 + lax.axis_index("c")
    base = wid * BPW
    pltpu.sync_copy(idxf.at[pl.ds(base, BPW)], idxvf)
    pltpu.sync_copy(idxd.at[pl.ds(base, BPW)], idxvd)
    work = ((akf, idxvf, gkf, 0), (avf, idxvf, gvf, 1),
            (akd, idxvd, gkd, 2), (avd, idxvd, gvd, 3))
    cps = [pltpu.async_copy(tab.at[idxv], rows.at[k], sem)
           for tab, idxv, out, k in work]
    for cp, (tab, idxv, out, k) in zip(cps, work):
        cp.wait()
        pltpu.sync_copy(rows.at[k], out.at[pl.ds(base, BPW)])


@functools.lru_cache(maxsize=None)
def _sc_gather_kernel():
    return pl.kernel(
        _sc_gather_body,
        out_type=[jax.ShapeDtypeStruct((NB, D), jnp.float32)] * 4,
        mesh=plsc.VectorSubcoreMesh(core_axis_name="c", subcore_axis_name="s",
                                    num_cores=2, num_subcores=16),
        scratch_types=[
            pltpu.VMEM((BPW,), jnp.int32),
            pltpu.VMEM((BPW,), jnp.int32),
            pltpu.VMEM((4, BPW, D), jnp.float32),
            pltpu.SemaphoreType.DMA,
        ],
    )


def _sc_gather(*args):
    return _sc_gather_kernel()(*args)


def _scatter_tc_body(idxf, idxd, rkf, rvf, rkd, rvd, akf, avf, akd, avd,
                     okf, ovf, okd, ovd, sem, *, nb):
    def scan(idxref, rk, rv, ok, ov):
        def it(i, c):
            r = idxref[i]
            pltpu.make_async_copy(rk.at[pl.ds(i, 1)],
                                  ok.at[pl.ds(r, 1)], sem).start()
            pltpu.make_async_copy(rv.at[pl.ds(i, 1)],
                                  ov.at[pl.ds(r, 1)], sem).start()
            return c
        lax.fori_loop(0, nb, it, jnp.int32(0))
    scan(idxf, rkf, rvf, okf, ovf)
    scan(idxd, rkd, rvd, okd, ovd)

    def drain(i, c):
        pltpu.make_async_copy(rkf.at[pl.ds(0, 1)],
                              okf.at[pl.ds(0, 1)], sem).wait()
        return c
    lax.fori_loop(0, 4 * nb, drain, jnp.int32(0))


def _sc_scatter(akf, avf, akd, avd, idxf, idxd, rkf, rvf, rkd, rvd):
    nb = rkf.shape[0]
    m = akf.shape[0]
    body = functools.partial(_scatter_tc_body, nb=nb)
    vspec = pl.BlockSpec((nb, D), lambda: (0, 0))
    ispec = pl.BlockSpec(memory_space=pltpu.SMEM)
    aspec = pl.BlockSpec(memory_space=pl.ANY)
    return pl.pallas_call(
        body,
        in_specs=[ispec, ispec, vspec, vspec, vspec, vspec,
                  aspec, aspec, aspec, aspec],
        out_specs=[aspec] * 4,
        out_shape=[jax.ShapeDtypeStruct((m, D), jnp.float32)] * 4,
        input_output_aliases={6: 0, 7: 1, 8: 2, 9: 3},
        scratch_shapes=[pltpu.SemaphoreType.DMA],
        compiler_params=pltpu.CompilerParams(
            vmem_limit_bytes=100 * 1024 * 1024),
    )(idxf, idxd, rkf, rvf, rkd, rvd, akf, avf, akd, avd)


def kernel(x, write_mask, W1_0, b1_0, W2_0, b2_0, g_0, be_0, W1_1, b1_1,
           W2_1, b2_1, g_1, be_1, Wf, bf, g_ln, b_ln, K_fast, V_fast,
           K_deep, V_deep):
    B, T, d = x.shape
    xf = x.reshape(B * T, d)
    h = _mamba(xf, W1_0, b1_0, W2_0, b2_0, g_0, be_0, W1_1, b1_1, W2_1,
               b2_1, g_1, be_1).reshape(B, T, d)

    anyb = jnp.any(write_mask, axis=0).astype(jnp.float32)
    invsq = jnp.float32(1.0 / math.sqrt(d))

    akf, avf, akd, avd = K_fast, V_fast, K_deep, V_deep
    p_t = jnp.float32(0.0)
    outs = []
    for t in range(T):
        h_t = h[:, t, :]
        decp = DECAY ** p_t
        scal = jnp.stack([decp, decp, invsq])
        vf, idxf = _select(h_t, akf, avf, scal)
        vd, idxd = _select(h_t, akd, avd, scal)
        if t + 1 < T:
            a_t = anyb[t]
            cscale = a_t / (decp * DECAY)
            uscal = jnp.stack([LR_FAST * a_t, LR_FAST * cscale,
                               LR_DEEP * a_t, LR_DEEP * cscale])
            gkf, gvf, gkd, gvd = _sc_gather(
                akf, avf, akd, avd, idxf.reshape(B), idxd.reshape(B))
            out_t, rkf, rvf, rkd, rvd = _fused_upd(
                h_t, vf, vd, Wf, bf, g_ln, b_ln, gkf, gvf, gkd, gvd,
                idxf, idxf.reshape(1, B), idxd, idxd.reshape(1, B), uscal)
            akf, avf, akd, avd = _sc_scatter(
                akf, avf, akd, avd, idxf.reshape(B), idxd.reshape(B),
                rkf, rvf, rkd, rvd)
            p_t = p_t + a_t
        else:
            out_t = _fused_only(h_t, vf, vd, Wf, bf, g_ln, b_ln)
        outs.append(out_t)
    return jnp.stack(outs, axis=1)
